# baseline (speedup 1.0000x reference)
import functools

import jax
import jax.numpy as jnp
from jax import lax
from jax.experimental import pallas as pl
from jax.experimental.pallas import tpu as pltpu
from jax.experimental.pallas import tpu_sc as plsc

_NC = 2
_NS = 16
_CHUNK = 128


@functools.lru_cache(maxsize=None)
def _make_gather(V, D, B):
    NW = _NC * _NS
    b_per_w = B // NW
    n_chunks = b_per_w // _CHUNK
    assert b_per_w * NW == B and n_chunks * _CHUNK == b_per_w
    mesh = plsc.VectorSubcoreMesh(core_axis_name="c", subcore_axis_name="s")

    UNROLL = 4

    @functools.partial(
        pl.kernel,
        mesh=mesh,
        out_type=[
            jax.ShapeDtypeStruct((B, D), jnp.float32),
            jax.ShapeDtypeStruct((B, D), jnp.float32),
        ],
        scratch_types=[
            pltpu.VMEM((b_per_w + 16,), jnp.int32),
            pltpu.VMEM((b_per_w + 16,), jnp.int32),
            pltpu.SemaphoreType.DMA,
        ],
    )
    def gather(table_hbm, nidx_hbm, midx_hbm, out_n, out_m,
               idx_n, idx_m, sem):
        wid = lax.axis_index("s") * _NC + lax.axis_index("c")
        base = wid * b_per_w
        pltpu.sync_copy(nidx_hbm.at[pl.ds(base, b_per_w)],
                        idx_n.at[pl.ds(0, b_per_w)])
        pltpu.sync_copy(midx_hbm.at[pl.ds(base, b_per_w)],
                        idx_m.at[pl.ds(0, b_per_w)])

        def body(i, _):
            for k in range(UNROLL):
                j = i * UNROLL + k
                rn = idx_n[pl.ds(j, 16)][0]
                rm = idx_m[pl.ds(j, 16)][0]
                pltpu.async_copy(table_hbm.at[rn], out_n.at[base + j], sem)
                pltpu.async_copy(table_hbm.at[rm], out_m.at[base + j], sem)
            return ()

        lax.fori_loop(0, b_per_w // UNROLL, body, ())
        pltpu.make_async_copy(out_n.at[pl.ds(base, b_per_w)],
                              out_n.at[pl.ds(base, b_per_w)], sem).wait()
        pltpu.make_async_copy(out_m.at[pl.ds(base, b_per_w)],
                              out_m.at[pl.ds(base, b_per_w)], sem).wait()

    return gather


def _score_body(nrel, node_ref, nbr_ref, rel_ref, rv_ref, out_ref):
    D = rv_ref.shape[1]
    node = node_ref[:, :D]
    nbr = nbr_ref[:, :D]
    rel = rel_ref[...]
    x = jnp.concatenate(
        [jnp.where(rel == r, node, 0.0) for r in range(nrel)], axis=1)
    t = lax.dot_general(x, rv_ref[...], (((1,), (0,)), ((), ())),
                        preferred_element_type=jnp.float32)
    score = jnp.sum(t * nbr, axis=1, keepdims=True)
    out_ref[...] = jax.nn.sigmoid(score)


@functools.lru_cache(maxsize=None)
def _make_score(B, D, R, Bb=1024, interpret=False):
    grid = (B // Bb,)
    return pl.pallas_call(
        functools.partial(_score_body, R),
        grid=grid,
        in_specs=[
            pl.BlockSpec((Bb, D), lambda i: (i, 0)),
            pl.BlockSpec((Bb, D), lambda i: (i, 0)),
            pl.BlockSpec((Bb, 1), lambda i: (i, 0)),
            pl.BlockSpec((R * D, D), lambda i: (0, 0)),
        ],
        out_specs=pl.BlockSpec((Bb, 1), lambda i: (i, 0)),
        out_shape=jax.ShapeDtypeStruct((B, 1), jnp.float32),
        interpret=interpret,
    )


def kernel(node_idx, relation_idx, node_neighbor_idx, node_embed_table,
           relation_embed_table):
    B = node_idx.shape[0]
    V, D = node_embed_table.shape
    R = relation_embed_table.shape[0]
    node_rows, nbr_rows = _make_gather(V, D, B)(
        node_embed_table, node_idx.astype(jnp.int32),
        node_neighbor_idx.astype(jnp.int32))
    rel2d = relation_idx.astype(jnp.int32).reshape(B, 1)
    rv = relation_embed_table.reshape(R * D, D)
    return _make_score(B, D, R)(node_rows, nbr_rows, rel2d, rv)


# speedup vs baseline: 4.7494x; 4.7494x over previous
import functools

import jax
import jax.numpy as jnp
from jax import lax
from jax.experimental import pallas as pl
from jax.experimental.pallas import tpu as pltpu
from jax.experimental.pallas import tpu_sc as plsc

_NC = 2
_NS = 16
_CHUNK = 128
_L = 16
_OH = 16


@functools.lru_cache(maxsize=None)
def _make_gather(V, D, B):
    NW = _NC * _NS
    b_per_w = B // NW
    n_chunks = b_per_w // _CHUNK
    assert b_per_w * NW == B and n_chunks * _CHUNK == b_per_w
    W = 2 * D
    mesh = plsc.VectorSubcoreMesh(core_axis_name="c", subcore_axis_name="s")

    @functools.partial(
        pl.kernel,
        mesh=mesh,
        compiler_params=pltpu.CompilerParams(use_tc_tiling_on_sc=False,
                                             needs_layout_passes=False),
        out_type=[
            jax.ShapeDtypeStruct((B, W), jnp.float32),
            jax.ShapeDtypeStruct((B, W), jnp.float32),
        ],
        scratch_types=[
            pltpu.VMEM((n_chunks, _CHUNK), jnp.int32),
            pltpu.VMEM((n_chunks, _CHUNK), jnp.int32),
            pltpu.VMEM((b_per_w,), jnp.int32),
            pltpu.VMEM((b_per_w, D), jnp.float32),
            pltpu.VMEM((b_per_w, D), jnp.float32),
            pltpu.VMEM((b_per_w, _OH), jnp.float32),
            pltpu.SemaphoreType.DMA,
        ],
    )
    def gather(table_hbm, nidx_hbm, midx_hbm, ridx_hbm, out_n, out_m,
               idx_n, idx_m, rel_v, rows_n, rows_m, oh, sem):
        wid = lax.axis_index("s") * _NC + lax.axis_index("c")
        base = wid * b_per_w
        pltpu.sync_copy(nidx_hbm.at[pl.ds(wid * n_chunks, n_chunks)], idx_n)
        pltpu.sync_copy(midx_hbm.at[pl.ds(wid * n_chunks, n_chunks)], idx_m)
        pltpu.sync_copy(ridx_hbm.at[pl.ds(base, b_per_w)], rel_v)
        copies = []
        for j in range(n_chunks):
            dst = pl.ds(j * _CHUNK, _CHUNK)
            copies.append(pltpu.async_copy(
                table_hbm.at[idx_n.at[j]], rows_n.at[dst], sem))
            copies.append(pltpu.async_copy(
                table_hbm.at[idx_m.at[j]], rows_m.at[dst], sem))

        zeros = jnp.zeros((_L,), jnp.float32)

        def zbody(i, _):
            oh[i, :] = zeros
            return ()

        lax.fori_loop(0, b_per_w, zbody, ())
        ones = jnp.full((_L,), 1.0, jnp.float32)
        iota = lax.iota(jnp.int32, _L)

        def sbody(i, _):
            rows = iota + i * _L
            cols = rel_v[pl.ds(i * _L, _L)]
            plsc.store_scatter(oh, [rows, cols], ones)
            return ()

        lax.fori_loop(0, b_per_w // _L, sbody, ())

        for c in copies:
            c.wait()
        rows = pl.ds(base, b_per_w)
        pltpu.sync_copy(rows_n, out_n.at[rows, pl.ds(0, D)])
        pltpu.sync_copy(oh, out_n.at[rows, pl.ds(D, _OH)])
        pltpu.sync_copy(rows_m, out_m.at[rows, pl.ds(0, D)])

    return gather


def _score_body(nrel, node_ref, nbr_ref, rv_ref, out_ref):
    D = rv_ref.shape[1]
    nw = node_ref[...]
    node = nw[:, :D]
    oh = nw[:, D:D + nrel]
    nbr = nbr_ref[:, :D]
    x = jnp.concatenate(
        [node * oh[:, r:r + 1] for r in range(nrel)], axis=1)
    t = lax.dot_general(x, rv_ref[...], (((1,), (0,)), ((), ())),
                        preferred_element_type=jnp.float32)
    p = t * nbr
    ones = jnp.ones((D, 1), jnp.float32)
    score = lax.dot_general(p, ones, (((1,), (0,)), ((), ())),
                            preferred_element_type=jnp.float32)
    out_ref[...] = jax.nn.sigmoid(score)


@functools.lru_cache(maxsize=None)
def _make_score(B, D, R, Bb=2048, interpret=False):
    grid = (B // Bb,)
    return pl.pallas_call(
        functools.partial(_score_body, R),
        grid=grid,
        in_specs=[
            pl.BlockSpec((Bb, 2 * D), lambda i: (i, 0)),
            pl.BlockSpec((Bb, 2 * D), lambda i: (i, 0)),
            pl.BlockSpec((R * D, D), lambda i: (0, 0)),
        ],
        out_specs=pl.BlockSpec((Bb, 1), lambda i: (i, 0)),
        out_shape=jax.ShapeDtypeStruct((B, 1), jnp.float32),
        interpret=interpret,
    )


def kernel(node_idx, relation_idx, node_neighbor_idx, node_embed_table,
           relation_embed_table):
    B = node_idx.shape[0]
    V, D = node_embed_table.shape
    R = relation_embed_table.shape[0]
    nidx2 = node_idx.astype(jnp.int32).reshape(-1, _CHUNK)
    midx2 = node_neighbor_idx.astype(jnp.int32).reshape(-1, _CHUNK)
    node_rows, nbr_rows = _make_gather(V, D, B)(
        node_embed_table, nidx2, midx2, relation_idx.astype(jnp.int32))
    rv = relation_embed_table.reshape(R * D, D)
    return _make_score(B, D, R)(node_rows, nbr_rows, rv)


# speedup vs baseline: 4.7660x; 1.0035x over previous
import functools

import jax
import jax.numpy as jnp
from jax import lax
from jax.experimental import pallas as pl
from jax.experimental.pallas import tpu as pltpu
from jax.experimental.pallas import tpu_sc as plsc

_NC = 2
_NS = 16
_CHUNK = 128
_L = 16
_OH = 16


@functools.lru_cache(maxsize=None)
def _make_gather(V, D, B):
    NW = _NC * _NS
    b_per_w = B // NW
    n_chunks = b_per_w // _CHUNK
    assert b_per_w * NW == B and n_chunks * _CHUNK == b_per_w
    W = 2 * D
    mesh = plsc.VectorSubcoreMesh(core_axis_name="c", subcore_axis_name="s")

    @functools.partial(
        pl.kernel,
        mesh=mesh,
        compiler_params=pltpu.CompilerParams(use_tc_tiling_on_sc=False,
                                             needs_layout_passes=False),
        out_type=[
            jax.ShapeDtypeStruct((B, W), jnp.float32),
            jax.ShapeDtypeStruct((B, W), jnp.float32),
        ],
        scratch_types=[
            pltpu.VMEM((b_per_w,), jnp.int32),
            pltpu.VMEM((b_per_w,), jnp.int32),
            pltpu.VMEM((b_per_w,), jnp.int32),
            pltpu.VMEM((b_per_w, D), jnp.float32),
            pltpu.VMEM((b_per_w, D), jnp.float32),
            pltpu.VMEM((b_per_w, _OH), jnp.float32),
            pltpu.SemaphoreType.DMA,
        ],
    )
    def gather(table_hbm, nidx_hbm, midx_hbm, ridx_hbm, out_n, out_m,
               idx_n, idx_m, rel_v, rows_n, rows_m, oh, sem):
        wid = lax.axis_index("s") * _NC + lax.axis_index("c")
        base = wid * b_per_w
        pltpu.sync_copy(nidx_hbm.at[pl.ds(base, b_per_w)], idx_n)
        pltpu.sync_copy(midx_hbm.at[pl.ds(base, b_per_w)], idx_m)
        pltpu.sync_copy(ridx_hbm.at[pl.ds(base, b_per_w)], rel_v)
        copies = []
        for j in range(n_chunks):
            sl = pl.ds(j * _CHUNK, _CHUNK)
            copies.append(pltpu.async_copy(
                table_hbm.at[idx_n.at[sl]], rows_n.at[sl], sem))
            copies.append(pltpu.async_copy(
                table_hbm.at[idx_m.at[sl]], rows_m.at[sl], sem))

        zeros = jnp.zeros((_L,), jnp.float32)

        def zbody(i, _):
            oh[i, :] = zeros
            return ()

        lax.fori_loop(0, b_per_w, zbody, ())
        ones = jnp.full((_L,), 1.0, jnp.float32)
        iota = lax.iota(jnp.int32, _L)

        def sbody(i, _):
            rows = iota + i * _L
            cols = rel_v[pl.ds(i * _L, _L)]
            plsc.store_scatter(oh, [rows, cols], ones)
            return ()

        lax.fori_loop(0, b_per_w // _L, sbody, ())

        for c in copies:
            c.wait()
        rows = pl.ds(base, b_per_w)
        pltpu.sync_copy(rows_n, out_n.at[rows, pl.ds(0, D)])
        pltpu.sync_copy(oh, out_n.at[rows, pl.ds(D, _OH)])
        pltpu.sync_copy(rows_m, out_m.at[rows, pl.ds(0, D)])

    return gather


def _score_body(nrel, node_ref, nbr_ref, rv_ref, out_ref):
    D = rv_ref.shape[1]
    nw = node_ref[...]
    node = nw[:, :D]
    oh = nw[:, D:D + nrel]
    nbr = nbr_ref[:, :D]
    x = jnp.concatenate(
        [node * oh[:, r:r + 1] for r in range(nrel)], axis=1)
    t = lax.dot_general(x, rv_ref[...], (((1,), (0,)), ((), ())),
                        preferred_element_type=jnp.float32)
    p = t * nbr
    ones = jnp.ones((D, 1), jnp.float32)
    score = lax.dot_general(p, ones, (((1,), (0,)), ((), ())),
                            preferred_element_type=jnp.float32)
    out_ref[...] = jax.nn.sigmoid(score)


@functools.lru_cache(maxsize=None)
def _make_score(B, D, R, Bb=2048, interpret=False):
    grid = (B // Bb,)
    return pl.pallas_call(
        functools.partial(_score_body, R),
        grid=grid,
        in_specs=[
            pl.BlockSpec((Bb, 2 * D), lambda i: (i, 0)),
            pl.BlockSpec((Bb, 2 * D), lambda i: (i, 0)),
            pl.BlockSpec((R * D, D), lambda i: (0, 0)),
        ],
        out_specs=pl.BlockSpec((Bb, 1), lambda i: (i, 0)),
        out_shape=jax.ShapeDtypeStruct((B, 1), jnp.float32),
        interpret=interpret,
    )


def kernel(node_idx, relation_idx, node_neighbor_idx, node_embed_table,
           relation_embed_table):
    B = node_idx.shape[0]
    V, D = node_embed_table.shape
    R = relation_embed_table.shape[0]
    node_rows, nbr_rows = _make_gather(V, D, B)(
        node_embed_table, node_idx.astype(jnp.int32),
        node_neighbor_idx.astype(jnp.int32), relation_idx.astype(jnp.int32))
    rv = relation_embed_table.reshape(R * D, D)
    return _make_score(B, D, R)(node_rows, nbr_rows, rv)


# speedup vs baseline: 7.0266x; 1.4743x over previous
import functools

import jax
import jax.numpy as jnp
from jax import lax
from jax.experimental import pallas as pl
from jax.experimental.pallas import tpu as pltpu
from jax.experimental.pallas import tpu_sc as plsc

_NC = 2
_NS = 16
_L = 16
_CHUNK = 128


def _pad_body(in_ref, out_ref):
    D = in_ref.shape[1]
    out_ref[:, :D] = in_ref[...]
    out_ref[:, D:] = in_ref[...]


@functools.lru_cache(maxsize=None)
def _make_pad(V, D, BV=5000):
    grid = (V // BV,)
    return pl.pallas_call(
        _pad_body,
        grid=grid,
        in_specs=[pl.BlockSpec((BV, D), lambda i: (i, 0))],
        out_specs=pl.BlockSpec((BV, 2 * D), lambda i: (i, 0)),
        out_shape=jax.ShapeDtypeStruct((V, 2 * D), jnp.float32),
    )


@functools.lru_cache(maxsize=None)
def _make_gather(V, D, B, R):
    NW = _NC * _NS
    b_per_w = B // NW
    n_rounds = b_per_w // _CHUNK
    assert b_per_w * NW == B and n_rounds * _CHUNK == b_per_w
    W = 2 * D
    mesh = plsc.VectorSubcoreMesh(core_axis_name="c", subcore_axis_name="s")

    @functools.partial(
        pl.kernel,
        mesh=mesh,
        compiler_params=pltpu.CompilerParams(needs_layout_passes=False),
        out_type=[
            jax.ShapeDtypeStruct((B, W), jnp.float32),
            jax.ShapeDtypeStruct((B, W), jnp.float32),
        ],
        scratch_types=[
            pltpu.VMEM((b_per_w,), jnp.int32),
            pltpu.VMEM((b_per_w,), jnp.int32),
            pltpu.VMEM((b_per_w,), jnp.int32),
            pltpu.VMEM((_CHUNK, W), jnp.float32),
            pltpu.VMEM((_CHUNK, W), jnp.float32),
            pltpu.VMEM((_CHUNK, W), jnp.float32),
            pltpu.VMEM((_CHUNK, W), jnp.float32),
            pltpu.SemaphoreType.DMA,
            pltpu.SemaphoreType.DMA,
        ],
    )
    def gather(table_hbm, nidx_hbm, midx_hbm, ridx_hbm, out_n, out_m,
               idx_n, idx_m, rel_v, sn0, sn1, sm0, sm1, sem, sem_out):
        wid = lax.axis_index("s") * _NC + lax.axis_index("c")
        base = wid * b_per_w
        pltpu.sync_copy(nidx_hbm.at[pl.ds(base, b_per_w)], idx_n)
        pltpu.sync_copy(midx_hbm.at[pl.ds(base, b_per_w)], idx_m)
        pltpu.sync_copy(ridx_hbm.at[pl.ds(base, b_per_w)], rel_v)
        slabs_n = (sn0, sn1)
        slabs_m = (sm0, sm1)
        zeros = jnp.zeros((_L,), jnp.float32)
        ones = jnp.full((_L,), 1.0, jnp.float32)
        iota = lax.iota(jnp.int32, _L)
        colbase = jnp.full((_L,), D, jnp.int32)

        def fire(h):
            sl = pl.ds(h * _CHUNK, _CHUNK)
            pltpu.async_copy(table_hbm.at[idx_n.at[sl]],
                             slabs_n[h % 2], sem)
            pltpu.async_copy(table_hbm.at[idx_m.at[sl]],
                             slabs_m[h % 2], sem)

        def drain_gather(h):
            pltpu.make_async_copy(table_hbm.at[pl.ds(0, _CHUNK)],
                                  slabs_n[h % 2], sem).wait()
            pltpu.make_async_copy(table_hbm.at[pl.ds(0, _CHUNK)],
                                  slabs_m[h % 2], sem).wait()

        def drain_wb():
            pltpu.make_async_copy(table_hbm.at[pl.ds(0, _CHUNK)],
                                  out_n.at[pl.ds(0, _CHUNK)], sem_out).wait()
            pltpu.make_async_copy(table_hbm.at[pl.ds(0, _CHUNK)],
                                  out_m.at[pl.ds(0, _CHUNK)], sem_out).wait()

        fire(0)
        for h in range(n_rounds):
            drain_gather(h)
            if h + 1 < n_rounds:
                if h >= 1:
                    drain_wb()
                fire(h + 1)
            sn = slabs_n[h % 2]
            for j in range(_CHUNK):
                sn[j, pl.ds(D, _L)] = zeros
            for j in range(_CHUNK // _L):
                rows = iota + j * _L
                cols = colbase + rel_v[pl.ds(h * _CHUNK + j * _L, _L)]
                plsc.store_scatter(sn, [rows, cols], ones)
            rows_out = pl.ds(base + h * _CHUNK, _CHUNK)
            pltpu.async_copy(sn, out_n.at[rows_out], sem_out)
            pltpu.async_copy(slabs_m[h % 2], out_m.at[rows_out], sem_out)
        for _ in range(min(2, n_rounds)):
            drain_wb()

    return gather


def _score_body(nrel, node_ref, nbr_ref, rv_ref, out_ref):
    D = rv_ref.shape[1]
    nw = node_ref[...]
    node = nw[:, :D]
    oh = nw[:, D:D + nrel]
    nbr = nbr_ref[:, :D]
    x = jnp.concatenate(
        [node * oh[:, r:r + 1] for r in range(nrel)], axis=1)
    t = lax.dot_general(x, rv_ref[...], (((1,), (0,)), ((), ())),
                        preferred_element_type=jnp.float32)
    p = t * nbr
    ones = jnp.ones((D, 1), jnp.float32)
    score = lax.dot_general(p, ones, (((1,), (0,)), ((), ())),
                            preferred_element_type=jnp.float32)
    out_ref[...] = jax.nn.sigmoid(score)


@functools.lru_cache(maxsize=None)
def _make_score(B, D, R, Bb=2048, interpret=False):
    grid = (B // Bb,)
    return pl.pallas_call(
        functools.partial(_score_body, R),
        grid=grid,
        in_specs=[
            pl.BlockSpec((Bb, 2 * D), lambda i: (i, 0)),
            pl.BlockSpec((Bb, 2 * D), lambda i: (i, 0)),
            pl.BlockSpec((R * D, D), lambda i: (0, 0)),
        ],
        out_specs=pl.BlockSpec((Bb, 1), lambda i: (i, 0)),
        out_shape=jax.ShapeDtypeStruct((B, 1), jnp.float32),
        interpret=interpret,
    )


def kernel(node_idx, relation_idx, node_neighbor_idx, node_embed_table,
           relation_embed_table):
    B = node_idx.shape[0]
    V, D = node_embed_table.shape
    R = relation_embed_table.shape[0]
    table2 = _make_pad(V, D)(node_embed_table)
    node_rows, nbr_rows = _make_gather(V, D, B, R)(
        table2, node_idx.astype(jnp.int32),
        node_neighbor_idx.astype(jnp.int32), relation_idx.astype(jnp.int32))
    rv = relation_embed_table.reshape(R * D, D)
    return _make_score(B, D, R)(node_rows, nbr_rows, rv)
